# speedup vs baseline: 3.0510x; 3.0510x over previous
import functools

import jax
import jax.numpy as jnp
from jax import lax
from jax.experimental import pallas as pl
from jax.experimental.pallas import tpu as pltpu
from jax.experimental.pallas import tpu_sc as plsc

NUM_CORES = 2
NUM_SUBCORES = 16
NUM_WORKERS = NUM_CORES * NUM_SUBCORES

CHUNK = 128


@functools.partial(jax.jit, static_argnames=("n_rows", "dim", "n_chunks"))
def _sc_gather(idx, weight, *, n_rows, dim, n_chunks):
    mesh = plsc.VectorSubcoreMesh(core_axis_name="c", subcore_axis_name="s")

    @functools.partial(
        pl.kernel,
        out_type=jax.ShapeDtypeStruct((n_rows, dim), jnp.float32),
        mesh=mesh,
        scratch_types=[
            pltpu.VMEM((n_chunks, CHUNK), jnp.int32),
            pltpu.VMEM((CHUNK, dim), jnp.float32),
            pltpu.SemaphoreType.DMA,
        ],
    )
    def k(idx_hbm, table_hbm, out_hbm, idx_v, rows_v, sem):
        wid = lax.axis_index("s") * NUM_CORES + lax.axis_index("c")
        base = wid * (n_chunks * CHUNK)
        pltpu.sync_copy(idx_hbm.at[wid], idx_v)

        def chunk_body(j, carry):
            pltpu.async_copy(table_hbm.at[idx_v.at[j]], rows_v, sem).wait()
            pltpu.sync_copy(rows_v, out_hbm.at[pl.ds(base + j * CHUNK, CHUNK)])
            return carry

        lax.fori_loop(0, n_chunks, chunk_body, 0, unroll=False)

    return k(idx, weight)


def kernel(input, weight):
    b, h = input.shape
    v, d = weight.shape
    n_rows = b * h
    per_worker = n_rows // NUM_WORKERS
    n_chunks = per_worker // CHUNK
    idx = input.reshape(NUM_WORKERS, n_chunks, CHUNK).astype(jnp.int32)
    out = _sc_gather(idx, weight, n_rows=n_rows, dim=d, n_chunks=n_chunks)
    return out.reshape(b, h, d)


# speedup vs baseline: 3.4447x; 1.1291x over previous
import functools

import jax
import jax.numpy as jnp
from jax import lax
from jax.experimental import pallas as pl
from jax.experimental.pallas import tpu as pltpu
from jax.experimental.pallas import tpu_sc as plsc

NUM_CORES = 2
NUM_SUBCORES = 16
NUM_WORKERS = NUM_CORES * NUM_SUBCORES

CHUNK = 128
NBUF = 4


@functools.partial(jax.jit, static_argnames=("n_rows", "dim", "n_chunks"))
def _sc_gather(idx, weight, *, n_rows, dim, n_chunks):
    mesh = plsc.VectorSubcoreMesh(core_axis_name="c", subcore_axis_name="s")
    n_groups = n_chunks // NBUF

    @functools.partial(
        pl.kernel,
        out_type=jax.ShapeDtypeStruct((n_rows, dim), jnp.float32),
        mesh=mesh,
        scratch_types=[
            pltpu.VMEM((n_chunks, CHUNK), jnp.int32),
            pltpu.VMEM((NBUF, CHUNK, dim), jnp.float32),
            [pltpu.SemaphoreType.DMA] * NBUF,
            [pltpu.SemaphoreType.DMA] * NBUF,
        ],
    )
    def k(idx_hbm, table_hbm, out_hbm, idx_v, rows_v, gsems, osems):
        wid = lax.axis_index("s") * NUM_CORES + lax.axis_index("c")
        base = wid * (n_chunks * CHUNK)

        pltpu.sync_copy(idx_hbm.at[wid], idx_v)

        def gather(j, b):
            pltpu.async_copy(table_hbm.at[idx_v.at[j]], rows_v.at[b], gsems[b])

        def wait_gather(b):
            pltpu.make_async_copy(
                table_hbm.at[idx_v.at[0]], rows_v.at[b], gsems[b]
            ).wait()

        def put(j, b):
            pltpu.async_copy(
                rows_v.at[b], out_hbm.at[pl.ds(base + j * CHUNK, CHUNK)], osems[b]
            )

        def wait_put(b):
            pltpu.make_async_copy(
                rows_v.at[b], out_hbm.at[pl.ds(base, CHUNK)], osems[b]
            ).wait()

        for b in range(NBUF):
            gather(b, b)

        def group_body(g, carry):
            for b in range(NBUF):
                wait_gather(b)
                put(g * NBUF + b, b)
            for b in range(NBUF):
                wait_put(b)
                gather((g + 1) * NBUF + b, b)
            return carry

        lax.fori_loop(0, n_groups - 1, group_body, 0, unroll=False)

        for b in range(NBUF):
            wait_gather(b)
            put((n_groups - 1) * NBUF + b, b)
        for b in range(NBUF):
            wait_put(b)

    return k(idx, weight)


def kernel(input, weight):
    b, h = input.shape
    v, d = weight.shape
    n_rows = b * h
    per_worker = n_rows // NUM_WORKERS
    n_chunks = per_worker // CHUNK
    idx = input.reshape(NUM_WORKERS, n_chunks, CHUNK).astype(jnp.int32)
    out = _sc_gather(idx, weight, n_rows=n_rows, dim=d, n_chunks=n_chunks)
    return out.reshape(b, h, d)


# speedup vs baseline: 3.4467x; 1.0006x over previous
import functools

import jax
import jax.numpy as jnp
from jax import lax
from jax.experimental import pallas as pl
from jax.experimental.pallas import tpu as pltpu
from jax.experimental.pallas import tpu_sc as plsc

NUM_CORES = 2
NUM_SUBCORES = 16
NUM_WORKERS = NUM_CORES * NUM_SUBCORES

CHUNK = 128
NBUF = 4


@functools.partial(jax.jit, static_argnames=("n_rows", "dim", "n_chunks"))
def _sc_gather(idx, weight, *, n_rows, dim, n_chunks):
    mesh = plsc.VectorSubcoreMesh(core_axis_name="c", subcore_axis_name="s")
    n_groups = n_chunks // NBUF

    @functools.partial(
        pl.kernel,
        out_type=jax.ShapeDtypeStruct((n_rows, dim), jnp.float32),
        mesh=mesh,
        scratch_types=[
            pltpu.VMEM((n_chunks, CHUNK), jnp.int32),
            pltpu.VMEM((NBUF, CHUNK, dim), jnp.float32),
            [pltpu.SemaphoreType.DMA] * NBUF,
            [pltpu.SemaphoreType.DMA] * NBUF,
        ],
        compiler_params=pltpu.CompilerParams(use_tc_tiling_on_sc=True),
    )
    def k(idx_hbm, table_hbm, out_hbm, idx_v, rows_v, gsems, osems):
        wid = lax.axis_index("s") * NUM_CORES + lax.axis_index("c")
        base = wid * (n_chunks * CHUNK)

        pltpu.sync_copy(idx_hbm.at[wid], idx_v)

        def gather(j, b):
            pltpu.async_copy(table_hbm.at[idx_v.at[j]], rows_v.at[b], gsems[b])

        def wait_gather(b):
            pltpu.make_async_copy(
                table_hbm.at[idx_v.at[0]], rows_v.at[b], gsems[b]
            ).wait()

        def put(j, b):
            pltpu.async_copy(
                rows_v.at[b], out_hbm.at[pl.ds(base + j * CHUNK, CHUNK)], osems[b]
            )

        def wait_put(b):
            pltpu.make_async_copy(
                rows_v.at[b], out_hbm.at[pl.ds(base, CHUNK)], osems[b]
            ).wait()

        for b in range(NBUF):
            gather(b, b)

        def group_body(g, carry):
            for b in range(NBUF):
                wait_gather(b)
                put(g * NBUF + b, b)
            for b in range(NBUF):
                wait_put(b)
                gather((g + 1) * NBUF + b, b)
            return carry

        lax.fori_loop(0, n_groups - 1, group_body, 0, unroll=False)

        for b in range(NBUF):
            wait_gather(b)
            put((n_groups - 1) * NBUF + b, b)
        for b in range(NBUF):
            wait_put(b)

    return k(idx, weight)


def kernel(input, weight):
    b, h = input.shape
    v, d = weight.shape
    n_rows = b * h
    per_worker = n_rows // NUM_WORKERS
    n_chunks = per_worker // CHUNK
    idx = input.reshape(NUM_WORKERS, n_chunks, CHUNK).astype(jnp.int32)
    out = _sc_gather(idx, weight, n_rows=n_rows, dim=d, n_chunks=n_chunks)
    return out.reshape(b, h, d)


# speedup vs baseline: 11.5783x; 3.3592x over previous
import functools

import jax
import jax.numpy as jnp
from jax import lax
from jax.experimental import pallas as pl
from jax.experimental.pallas import tpu as pltpu
from jax.experimental.pallas import tpu_sc as plsc

NUM_CORES = 2
NUM_SUBCORES = 16
NUM_WORKERS = NUM_CORES * NUM_SUBCORES

CHUNK = 128
NBUF = 4


@functools.partial(jax.jit, static_argnames=("n_rows", "dim", "n_chunks"))
def _sc_gather(idx, weight, *, n_rows, dim, n_chunks):
    mesh = plsc.VectorSubcoreMesh(core_axis_name="c", subcore_axis_name="s")
    n_groups = n_chunks // NBUF

    @functools.partial(
        pl.kernel,
        out_type=jax.ShapeDtypeStruct((n_rows, dim), jnp.float32),
        mesh=mesh,
        scratch_types=[
            pltpu.VMEM((n_chunks, CHUNK), jnp.int32),
            pltpu.VMEM((NBUF, CHUNK, dim), jnp.float32),
            [pltpu.SemaphoreType.DMA] * NBUF,
            [pltpu.SemaphoreType.DMA] * NBUF,
        ],
        compiler_params=pltpu.CompilerParams(use_tc_tiling_on_sc=True),
    )
    def k(idx_hbm, table_hbm, out_hbm, idx_v, rows_v, gsems, osems):
        wid = lax.axis_index("s") * NUM_CORES + lax.axis_index("c")
        base = wid * (n_chunks * CHUNK)

        pltpu.sync_copy(idx_hbm.at[wid], idx_v)

        def gather(j, b):
            pltpu.async_copy(table_hbm.at[idx_v.at[j]], rows_v.at[b], gsems[b])

        def wait_gather(b):
            pltpu.make_async_copy(
                table_hbm.at[idx_v.at[0]], rows_v.at[b], gsems[b]
            ).wait()

        def put(j, b):
            pltpu.async_copy(
                rows_v.at[b], out_hbm.at[pl.ds(base + j * CHUNK, CHUNK)], osems[b]
            )

        def wait_put(b):
            pltpu.make_async_copy(
                rows_v.at[b], out_hbm.at[pl.ds(base, CHUNK)], osems[b]
            ).wait()

        for b in range(NBUF):
            gather(b, b)

        def group_body(g, carry):
            for b in range(NBUF):
                wait_gather(b)
                put(g * NBUF + b, b)
            for b in range(NBUF):
                wait_put(b)
                gather((g + 1) * NBUF + b, b)
            return carry

        lax.fori_loop(0, n_groups - 1, group_body, 0, unroll=False)

        for b in range(NBUF):
            wait_gather(b)
            put((n_groups - 1) * NBUF + b, b)
        for b in range(NBUF):
            wait_put(b)

    return k(idx, weight)


def kernel(input, weight):
    b, h = input.shape
    v, d = weight.shape
    n_rows = b * h
    per_worker = n_rows // NUM_WORKERS
    n_chunks = per_worker // CHUNK
    idx = input.T.reshape(NUM_WORKERS, n_chunks, CHUNK).astype(jnp.int32)
    out = _sc_gather(idx, weight, n_rows=n_rows, dim=d, n_chunks=n_chunks)
    return out.reshape(h, b, d).transpose(1, 0, 2)


# speedup vs baseline: 11.8430x; 1.0229x over previous
import functools

import jax
import jax.numpy as jnp
from jax import lax
from jax.experimental import pallas as pl
from jax.experimental.pallas import tpu as pltpu
from jax.experimental.pallas import tpu_sc as plsc

NUM_CORES = 2
NUM_SUBCORES = 16
NUM_WORKERS = NUM_CORES * NUM_SUBCORES

CHUNK = 128
NBUF = 5


@functools.partial(jax.jit, static_argnames=("n_rows", "dim", "n_chunks"))
def _sc_gather(idx, weight, *, n_rows, dim, n_chunks):
    mesh = plsc.VectorSubcoreMesh(core_axis_name="c", subcore_axis_name="s")
    n_groups = n_chunks // NBUF

    @functools.partial(
        pl.kernel,
        out_type=jax.ShapeDtypeStruct((n_rows, dim), jnp.float32),
        mesh=mesh,
        scratch_types=[
            pltpu.VMEM((n_chunks, CHUNK), jnp.int32),
            pltpu.VMEM((NBUF, CHUNK, dim), jnp.float32),
            [pltpu.SemaphoreType.DMA] * NBUF,
            [pltpu.SemaphoreType.DMA] * NBUF,
        ],
        compiler_params=pltpu.CompilerParams(use_tc_tiling_on_sc=True),
    )
    def k(idx_hbm, table_hbm, out_hbm, idx_v, rows_v, gsems, osems):
        wid = lax.axis_index("s") * NUM_CORES + lax.axis_index("c")
        base = wid * (n_chunks * CHUNK)

        pltpu.sync_copy(idx_hbm.at[wid], idx_v)

        def gather(j, b):
            pltpu.async_copy(table_hbm.at[idx_v.at[j]], rows_v.at[b], gsems[b])

        def wait_gather(b):
            pltpu.make_async_copy(
                table_hbm.at[idx_v.at[0]], rows_v.at[b], gsems[b]
            ).wait()

        def put(j, b):
            pltpu.async_copy(
                rows_v.at[b], out_hbm.at[pl.ds(base + j * CHUNK, CHUNK)], osems[b]
            )

        def wait_put(b):
            pltpu.make_async_copy(
                rows_v.at[b], out_hbm.at[pl.ds(base, CHUNK)], osems[b]
            ).wait()

        for b in range(NBUF):
            gather(b, b)

        def group_body(g, carry):
            for b in range(NBUF):
                wait_gather(b)
                put(g * NBUF + b, b)
            for b in range(NBUF):
                wait_put(b)
                gather((g + 1) * NBUF + b, b)
            return carry

        lax.fori_loop(0, n_groups - 1, group_body, 0, unroll=False)

        for b in range(NBUF):
            wait_gather(b)
            put((n_groups - 1) * NBUF + b, b)
        for b in range(NBUF):
            wait_put(b)

    return k(idx, weight)


def kernel(input, weight):
    b, h = input.shape
    v, d = weight.shape
    n_rows = b * h
    per_worker = n_rows // NUM_WORKERS
    n_chunks = per_worker // CHUNK
    idx = input.T.reshape(NUM_WORKERS, n_chunks, CHUNK).astype(jnp.int32)
    out = _sc_gather(idx, weight, n_rows=n_rows, dim=d, n_chunks=n_chunks)
    return out.reshape(h, b, d).transpose(1, 0, 2)
